# Initial kernel scaffold; baseline (speedup 1.0000x reference)
import jax
import jax.numpy as jnp
from jax.experimental import pallas as pl


def kernel(input_ids, emb):
    raise NotImplementedError("write your pallas kernel here")


# speedup vs baseline: 5.1338x; 5.1338x over previous
import functools

import jax
import jax.numpy as jnp
from jax import lax
from jax.experimental import pallas as pl
from jax.experimental.pallas import tpu as pltpu
from jax.experimental.pallas import tpu_sc as plsc

B, S, V, D = 16384, 200, 32, 4
N = B * S
_info = plsc.get_sparse_core_info()
NC, NS, L = _info.num_cores, _info.num_subcores, _info.num_lanes
NW = NC * NS
PER_W = N // NW
CHUNK = 12800
NCHUNK = PER_W // CHUNK
GROUPS = CHUNK // L

_mesh = plsc.VectorSubcoreMesh(core_axis_name="c", subcore_axis_name="s")


@functools.partial(
    pl.kernel,
    mesh=_mesh,
    out_type=jax.ShapeDtypeStruct((N * D,), jnp.float32),
    scratch_types=[
        pltpu.VMEM((V * D,), jnp.float32),
        pltpu.VMEM((CHUNK,), jnp.int32),
        pltpu.VMEM((CHUNK * D,), jnp.float32),
    ],
    compiler_params=pltpu.CompilerParams(needs_layout_passes=False),
)
def _emb_lookup(tab_hbm, ids_hbm, out_hbm, tab_v, idx_v, out_v):
    wid = lax.axis_index("s") * NC + lax.axis_index("c")
    base = wid * PER_W
    pltpu.sync_copy(tab_hbm, tab_v)
    iota4 = lax.iota(jnp.int32, L) * D

    def chunk_body(ci, _):
        pltpu.sync_copy(ids_hbm.at[pl.ds(base + ci * CHUNK, CHUNK)], idx_v)

        def group_body(g, _):
            idx = idx_v[pl.ds(g * L, L)]
            src = idx * D
            pos = iota4 + g * (L * D)
            for k in range(D):
                ck = plsc.load_gather(tab_v, [src + k])
                plsc.store_scatter(out_v, [pos + k], ck)
            return 0

        lax.fori_loop(0, GROUPS, group_body, 0, unroll=2)
        pltpu.sync_copy(
            out_v, out_hbm.at[pl.ds((base + ci * CHUNK) * D, CHUNK * D)]
        )
        return 0

    lax.fori_loop(0, NCHUNK, chunk_body, 0)


def kernel(input_ids, emb):
    out = _emb_lookup(emb.reshape(-1), input_ids.reshape(-1))
    return out.reshape(B, S, D)


# speedup vs baseline: 5.4826x; 1.0680x over previous
import functools

import jax
import jax.numpy as jnp
from jax import lax
from jax.experimental import pallas as pl
from jax.experimental.pallas import tpu as pltpu
from jax.experimental.pallas import tpu_sc as plsc

B, S, V, D = 16384, 200, 32, 4
N = B * S
_info = plsc.get_sparse_core_info()
NC, NS, L = _info.num_cores, _info.num_subcores, _info.num_lanes
NW = NC * NS
PER_W = N // NW
CHUNK = 10240
NCHUNK = PER_W // CHUNK

_mesh = plsc.VectorSubcoreMesh(core_axis_name="c", subcore_axis_name="s")


@functools.partial(
    pl.kernel,
    mesh=_mesh,
    out_type=jax.ShapeDtypeStruct((N * D,), jnp.float32),
    scratch_types=[
        pltpu.VMEM((V * D,), jnp.float32),
        pltpu.VMEM((CHUNK,), jnp.int32),
        pltpu.VMEM((CHUNK,), jnp.int32),
        pltpu.VMEM((CHUNK * D,), jnp.float32),
        pltpu.VMEM((CHUNK * D,), jnp.float32),
        pltpu.SemaphoreType.DMA,
        pltpu.SemaphoreType.DMA,
        pltpu.SemaphoreType.DMA,
        pltpu.SemaphoreType.DMA,
    ],
    compiler_params=pltpu.CompilerParams(needs_layout_passes=False),
)
def _emb_lookup(tab_hbm, ids_hbm, out_hbm, tab_v, idx_v0, idx_v1,
                out_v0, out_v1, in_sem0, in_sem1, out_sem0, out_sem1):
    wid = lax.axis_index("s") * NC + lax.axis_index("c")
    base = wid * PER_W
    pltpu.sync_copy(tab_hbm, tab_v)
    iota4 = lax.iota(jnp.int32, L) * D
    idx_bufs = (idx_v0, idx_v1)
    out_bufs = (out_v0, out_v1)
    in_sems = (in_sem0, in_sem1)
    out_sems = (out_sem0, out_sem1)

    def start_in(ci):
        return pltpu.async_copy(
            ids_hbm.at[pl.ds(base + ci * CHUNK, CHUNK)],
            idx_bufs[ci % 2], in_sems[ci % 2])

    def start_out(ci):
        return pltpu.async_copy(
            out_bufs[ci % 2],
            out_hbm.at[pl.ds((base + ci * CHUNK) * D, CHUNK * D)],
            out_sems[ci % 2])

    in_copies = [None] * NCHUNK
    out_copies = [None] * NCHUNK
    in_copies[0] = start_in(0)
    for ci in range(NCHUNK):
        if ci + 1 < NCHUNK:
            in_copies[ci + 1] = start_in(ci + 1)
        in_copies[ci].wait()
        idx_buf = idx_bufs[ci % 2]
        out_buf = out_bufs[ci % 2]
        if ci >= 2:
            out_copies[ci - 2].wait()

        @plsc.parallel_loop(0, CHUNK, L, unroll=8)
        def group_body(off):
            idx = idx_buf[pl.ds(off, L)]
            src = idx * D
            pos = iota4 + off * D
            for k in range(D):
                ck = plsc.load_gather(tab_v, [src + k])
                plsc.store_scatter(out_buf, [pos + k], ck)

        out_copies[ci] = start_out(ci)
    out_copies[NCHUNK - 2].wait()
    out_copies[NCHUNK - 1].wait()


def kernel(input_ids, emb):
    out = _emb_lookup(emb.reshape(-1), input_ids.reshape(-1))
    return out.reshape(B, S, D)


# speedup vs baseline: 208.6893x; 38.0640x over previous
import functools

import jax
import jax.numpy as jnp
from jax import lax
from jax.experimental import pallas as pl
from jax.experimental.pallas import tpu as pltpu
from jax.experimental.pallas import tpu_sc as plsc

B, S, V, D = 16384, 200, 32, 4
N = B * S
_info = plsc.get_sparse_core_info()
NC, NS, L = _info.num_cores, _info.num_subcores, _info.num_lanes
NW = NC * NS
NJB = S // 8
NIB = B // 128
IB_PER_W = NIB // NW
IN_BLK = IB_PER_W * 8 * 128
OUT_BLK = 8 * IB_PER_W * D * 128
GROUPS = IN_BLK // L

_mesh = plsc.VectorSubcoreMesh(core_axis_name="c", subcore_axis_name="s")


@functools.partial(
    pl.kernel,
    mesh=_mesh,
    out_type=jax.ShapeDtypeStruct((S, B // 128 * D, 128), jnp.float32),
    scratch_types=[
        pltpu.VMEM((V * D,), jnp.float32),
        pltpu.VMEM((IB_PER_W * 8, 128), jnp.int32),
        pltpu.VMEM((IB_PER_W * 8, 128), jnp.int32),
        pltpu.VMEM((8, IB_PER_W * D, 128), jnp.float32),
        pltpu.VMEM((8, IB_PER_W * D, 128), jnp.float32),
        pltpu.SemaphoreType.DMA,
        pltpu.SemaphoreType.DMA,
        pltpu.SemaphoreType.DMA,
        pltpu.SemaphoreType.DMA,
    ],
    compiler_params=pltpu.CompilerParams(needs_layout_passes=False),
)
def _emb_lookup(tab_hbm, ids_hbm, out_hbm, tab_v, idx_v0, idx_v1,
                out_v0, out_v1, in_sem0, in_sem1, out_sem0, out_sem1):
    wid = lax.axis_index("s") * NC + lax.axis_index("c")
    ib0 = wid * IB_PER_W
    pltpu.sync_copy(tab_hbm, tab_v)
    idx_bufs = (idx_v0, idx_v1)
    out_bufs = (out_v0, out_v1)
    in_sems = (in_sem0, in_sem1)
    out_sems = (out_sem0, out_sem1)

    def start_in(jb):
        return pltpu.async_copy(
            ids_hbm.at[jb, pl.ds(ib0 * 8, IB_PER_W * 8)],
            idx_bufs[jb % 2], in_sems[jb % 2])

    def start_out(jb):
        return pltpu.async_copy(
            out_bufs[jb % 2],
            out_hbm.at[pl.ds(jb * 8, 8), pl.ds(ib0 * D, IB_PER_W * D)],
            out_sems[jb % 2])

    in_copies = [None] * NJB
    out_copies = [None] * NJB
    in_copies[0] = start_in(0)
    for jb in range(NJB):
        if jb + 1 < NJB:
            in_copies[jb + 1] = start_in(jb + 1)
        in_copies[jb].wait()
        idx_buf = idx_bufs[jb % 2]
        out_buf = out_bufs[jb % 2]
        if jb >= 2:
            out_copies[jb - 2].wait()

        @plsc.parallel_loop(0, GROUPS, 1, unroll=4)
        def group_body(g):
            r = g >> 3
            t = g & 7
            idx = idx_buf[r, pl.ds(t * L, L)]
            src = idx * D
            a = r & 7
            ib_l = r >> 3
            for d in range(D):
                ck = plsc.load_gather(tab_v, [src + d])
                out_buf[a, ib_l * D + d, pl.ds(t * L, L)] = ck

        out_copies[jb] = start_out(jb)
    out_copies[NJB - 2].wait()
    out_copies[NJB - 1].wait()


def kernel(input_ids, emb):
    ids_phys = (
        input_ids.reshape(128, 128, 25, 8)
        .transpose(2, 0, 3, 1)
        .reshape(25, 1024, 128)
    )
    out_phys = _emb_lookup(emb.reshape(-1), ids_phys)
    return (
        out_phys.reshape(200, 128, 4, 128)
        .transpose(1, 3, 0, 2)
        .reshape(16384, 200, 4)
    )


# speedup vs baseline: 210.5278x; 1.0088x over previous
import functools

import jax
import jax.numpy as jnp
from jax import lax
from jax.experimental import pallas as pl
from jax.experimental.pallas import tpu as pltpu
from jax.experimental.pallas import tpu_sc as plsc

B, S, V, D = 16384, 200, 32, 4
N = B * S
_info = plsc.get_sparse_core_info()
NC, NS, L = _info.num_cores, _info.num_subcores, _info.num_lanes
NW = NC * NS
NJB = S // 8
NIB = B // 128
IB_PER_W = NIB // NW
IN_BLK = IB_PER_W * 8 * 128
OUT_BLK = 8 * IB_PER_W * D * 128
GROUPS = IN_BLK // L

_mesh = plsc.VectorSubcoreMesh(core_axis_name="c", subcore_axis_name="s")


@functools.partial(
    pl.kernel,
    mesh=_mesh,
    out_type=jax.ShapeDtypeStruct((S, B // 128 * D, 128), jnp.float32),
    scratch_types=[
        pltpu.VMEM((V * D,), jnp.float32),
        pltpu.VMEM((IB_PER_W * 8, 128), jnp.int32),
        pltpu.VMEM((IB_PER_W * 8, 128), jnp.int32),
        pltpu.VMEM((8, IB_PER_W * D, 128), jnp.float32),
        pltpu.VMEM((8, IB_PER_W * D, 128), jnp.float32),
        pltpu.SemaphoreType.DMA,
        pltpu.SemaphoreType.DMA,
        pltpu.SemaphoreType.DMA,
        pltpu.SemaphoreType.DMA,
    ],
    compiler_params=pltpu.CompilerParams(needs_layout_passes=False),
)
def _emb_lookup(tab_hbm, ids_hbm, out_hbm, tab_v, idx_v0, idx_v1,
                out_v0, out_v1, in_sem0, in_sem1, out_sem0, out_sem1):
    wid = lax.axis_index("s") * NC + lax.axis_index("c")
    ib0 = wid * IB_PER_W
    idx_bufs = (idx_v0, idx_v1)
    out_bufs = (out_v0, out_v1)
    in_sems = (in_sem0, in_sem1)
    out_sems = (out_sem0, out_sem1)

    def start_in(jb):
        return pltpu.async_copy(
            ids_hbm.at[jb, pl.ds(ib0 * 8, IB_PER_W * 8)],
            idx_bufs[jb % 2], in_sems[jb % 2])

    def start_out(jb):
        return pltpu.async_copy(
            out_bufs[jb % 2],
            out_hbm.at[pl.ds(jb * 8, 8), pl.ds(ib0 * D, IB_PER_W * D)],
            out_sems[jb % 2])

    in_copies = [None] * NJB
    out_copies = [None] * NJB
    in_copies[0] = start_in(0)
    pltpu.sync_copy(tab_hbm, tab_v)
    for jb in range(NJB):
        if jb + 1 < NJB:
            in_copies[jb + 1] = start_in(jb + 1)
        in_copies[jb].wait()
        idx_buf = idx_bufs[jb % 2]
        out_buf = out_bufs[jb % 2]
        if jb >= 2:
            out_copies[jb - 2].wait()

        @plsc.parallel_loop(0, GROUPS, 1, unroll=4)
        def group_body(g):
            r = g >> 3
            t = g & 7
            idx = idx_buf[r, pl.ds(t * L, L)]
            src = idx * D
            a = r & 7
            ib_l = r >> 3
            for d in range(D):
                ck = plsc.load_gather(tab_v, [src + d])
                out_buf[a, ib_l * D + d, pl.ds(t * L, L)] = ck

        out_copies[jb] = start_out(jb)
    out_copies[NJB - 2].wait()
    out_copies[NJB - 1].wait()


def kernel(input_ids, emb):
    ids_phys = (
        input_ids.reshape(128, 128, 25, 8)
        .transpose(2, 0, 3, 1)
        .reshape(25, 1024, 128)
    )
    out_phys = _emb_lookup(emb.reshape(-1), ids_phys)
    return (
        out_phys.reshape(200, 128, 4, 128)
        .transpose(1, 3, 0, 2)
        .reshape(16384, 200, 4)
    )
